# baseline (speedup 1.0000x reference)
import jax
import jax.numpy as jnp
from jax import lax
from jax.experimental import pallas as pl
from jax.experimental.pallas import tpu as pltpu
from jax.experimental.pallas import tpu_sc as plsc

VOCAB = 1000
EMBED = 16
BITS = 32
BATCH = 1024
OUT_COLS = EMBED * BITS
SLICE = 128
NSLICE = VOCAB * EMBED // SLICE
NC = 2
NS = 16
L = 16
NW = NC * NS
B_PER_W = BATCH // NW

_ONE_F32_BITS = 0x3F800000


def _sc_body(ids_hbm, table_hbm, out_hbm, idx_v, slice_v, rows_v, outbuf_v, sem):
    wid = lax.axis_index("s") * NC + lax.axis_index("c")
    base = wid * B_PER_W
    pltpu.sync_copy(ids_hbm.at[pl.ds(base, B_PER_W)], idx_v)

    lanes = lax.iota(jnp.int32, L)
    for g in range(B_PER_W // L):
        ids16 = idx_v[pl.ds(g * L, L)]
        slice_v[pl.ds(g * L, L)] = ids16 >> 3
    pltpu.async_copy(table_hbm.at[slice_v], rows_v, sem).wait()

    plsc.subcore_barrier()

    sh_hi = lanes
    sh_lo = lanes + L
    one_bits = jnp.full((L,), _ONE_F32_BITS, jnp.int32)

    @plsc.parallel_loop(
        jnp.int32(0), jnp.int32(B_PER_W), step=jnp.int32(1), unroll=1
    )
    def token_body(t):
        t_vec = jnp.full((L,), t, jnp.int32)
        tok = plsc.load_gather(idx_v, [t_vec])
        row = plsc.bitcast(
            plsc.load_gather(rows_v, [t_vec, (tok & 7) * EMBED + lanes]),
            jnp.int32,
        )
        for e in range(EMBED):
            v = row.at[jnp.full((L,), e, jnp.int32)].get(mode="promise_in_bounds")
            hi = ((v << sh_hi) >> 31) & one_bits
            lo = ((v << sh_lo) >> 31) & one_bits
            outbuf_v[t, pl.ds(e * BITS, L)] = plsc.bitcast(hi, jnp.float32)
            outbuf_v[t, pl.ds(e * BITS + L, L)] = plsc.bitcast(lo, jnp.float32)

    pltpu.sync_copy(outbuf_v, out_hbm.at[pl.ds(base, B_PER_W)])


def _build(interpret=False):
    mesh = plsc.VectorSubcoreMesh(core_axis_name="c", subcore_axis_name="s")
    return pl.kernel(
        _sc_body,
        out_type=jax.ShapeDtypeStruct((BATCH, OUT_COLS), jnp.float32),
        mesh=mesh,
        scratch_types=[
            pltpu.VMEM((B_PER_W,), jnp.int32),
            pltpu.VMEM((B_PER_W,), jnp.int32),
            pltpu.VMEM((B_PER_W, SLICE), jnp.float32),
            pltpu.VMEM((B_PER_W, OUT_COLS), jnp.float32),
            pltpu.SemaphoreType.DMA,
        ],
        compiler_params=pltpu.CompilerParams(
            needs_layout_passes=False,
            disable_bounds_checks=True,
        ),
        interpret=interpret,
    )


def kernel(token_ids, weight_float):
    ids32 = token_ids.astype(jnp.int32)
    out = _build()(ids32, weight_float.reshape(NSLICE, SLICE))
    return out.reshape(BATCH, EMBED, BITS)


# speedup vs baseline: 1.0061x; 1.0061x over previous
import jax
import jax.numpy as jnp
from jax import lax
from jax.experimental import pallas as pl
from jax.experimental.pallas import tpu as pltpu
from jax.experimental.pallas import tpu_sc as plsc

VOCAB = 1000
EMBED = 16
BITS = 32
BATCH = 1024
OUT_COLS = EMBED * BITS
SLICE = 128
NSLICE = VOCAB * EMBED // SLICE
NC = 2
NS = 16
L = 16
NW = NC * NS
B_PER_W = BATCH // NW

_ONE_F32_BITS = 0x3F800000


def _sc_body(ids_hbm, table_hbm, out_hbm, idx_v, slice_v, rows_v, outbuf_v, sem):
    wid = lax.axis_index("s") * NC + lax.axis_index("c")
    base = wid * B_PER_W
    pltpu.sync_copy(ids_hbm.at[pl.ds(base, B_PER_W)], idx_v)

    lanes = lax.iota(jnp.int32, L)
    for g in range(B_PER_W // L):
        ids16 = idx_v[pl.ds(g * L, L)]
        slice_v[pl.ds(g * L, L)] = ids16 >> 3
    pltpu.async_copy(table_hbm.at[slice_v], rows_v, sem).wait()

    plsc.subcore_barrier()

    sh_hi = lanes
    sh_lo = lanes + L
    one_bits = jnp.full((L,), _ONE_F32_BITS, jnp.int32)

    @plsc.parallel_loop(
        jnp.int32(0), jnp.int32(B_PER_W), step=jnp.int32(1), unroll=4
    )
    def token_body(t):
        t_vec = jnp.full((L,), t, jnp.int32)
        tok = plsc.load_gather(idx_v, [t_vec])
        row = plsc.bitcast(
            plsc.load_gather(rows_v, [t_vec, (tok & 7) * EMBED + lanes]),
            jnp.int32,
        )
        for e in range(EMBED):
            v = row.at[jnp.full((L,), e, jnp.int32)].get(mode="promise_in_bounds")
            hi = ((v << sh_hi) >> 31) & one_bits
            lo = ((v << sh_lo) >> 31) & one_bits
            outbuf_v[t, pl.ds(e * BITS, L)] = plsc.bitcast(hi, jnp.float32)
            outbuf_v[t, pl.ds(e * BITS + L, L)] = plsc.bitcast(lo, jnp.float32)

    pltpu.sync_copy(outbuf_v, out_hbm.at[pl.ds(base, B_PER_W)])


def _build(interpret=False):
    mesh = plsc.VectorSubcoreMesh(core_axis_name="c", subcore_axis_name="s")
    return pl.kernel(
        _sc_body,
        out_type=jax.ShapeDtypeStruct((BATCH, OUT_COLS), jnp.float32),
        mesh=mesh,
        scratch_types=[
            pltpu.VMEM((B_PER_W,), jnp.int32),
            pltpu.VMEM((B_PER_W,), jnp.int32),
            pltpu.VMEM((B_PER_W, SLICE), jnp.float32),
            pltpu.VMEM((B_PER_W, OUT_COLS), jnp.float32),
            pltpu.SemaphoreType.DMA,
        ],
        compiler_params=pltpu.CompilerParams(
            needs_layout_passes=False,
            disable_bounds_checks=True,
        ),
        interpret=interpret,
    )


def kernel(token_ids, weight_float):
    ids32 = token_ids.astype(jnp.int32)
    out = _build()(ids32, weight_float.reshape(NSLICE, SLICE))
    return out.reshape(BATCH, EMBED, BITS)


# speedup vs baseline: 1.0073x; 1.0013x over previous
import jax
import jax.numpy as jnp
from jax import lax
from jax.experimental import pallas as pl
from jax.experimental.pallas import tpu as pltpu
from jax.experimental.pallas import tpu_sc as plsc

VOCAB = 1000
EMBED = 16
BITS = 32
BATCH = 1024
OUT_COLS = EMBED * BITS
SLICE = 128
NSLICE = VOCAB * EMBED // SLICE
NC = 2
NS = 16
L = 16
NW = NC * NS
B_PER_W = BATCH // NW

_ONE_F32_BITS = 0x3F800000


def _sc_body(ids_hbm, table_hbm, out_hbm, idx_v, slice_v, rows_v, outbuf_v, sem):
    wid = lax.axis_index("s") * NC + lax.axis_index("c")
    base = wid * B_PER_W
    pltpu.sync_copy(ids_hbm.at[pl.ds(base, B_PER_W)], idx_v)

    lanes = lax.iota(jnp.int32, L)
    for g in range(B_PER_W // L):
        ids16 = idx_v[pl.ds(g * L, L)]
        slice_v[pl.ds(g * L, L)] = ids16 >> 3
    pltpu.async_copy(table_hbm.at[slice_v], rows_v, sem).wait()

    plsc.subcore_barrier()

    sh_hi = lanes
    sh_lo = lanes + L
    one_bits = jnp.full((L,), _ONE_F32_BITS, jnp.int32)

    @plsc.parallel_loop(
        jnp.int32(0), jnp.int32(B_PER_W), step=jnp.int32(1), unroll=4
    )
    def token_body(t):
        t_vec = jnp.full((L,), t, jnp.int32)
        tok = plsc.load_gather(idx_v, [t_vec])
        row = plsc.bitcast(
            plsc.load_gather(rows_v, [t_vec, (tok & 7) * EMBED + lanes]),
            jnp.int32,
        )
        for e in range(EMBED):
            v = row.at[jnp.full((L,), e, jnp.int32)].get(mode="promise_in_bounds")
            hi = ((v << sh_hi) >> 31) & one_bits
            lo = ((v << sh_lo) >> 31) & one_bits
            outbuf_v[t, pl.ds(e * BITS, L)] = plsc.bitcast(hi, jnp.float32)
            outbuf_v[t, pl.ds(e * BITS + L, L)] = plsc.bitcast(lo, jnp.float32)

    pltpu.sync_copy(outbuf_v, out_hbm.at[pl.ds(base, B_PER_W)])


def _build():
    mesh = plsc.VectorSubcoreMesh(core_axis_name="c", subcore_axis_name="s")
    return pl.kernel(
        _sc_body,
        out_type=jax.ShapeDtypeStruct((BATCH, OUT_COLS), jnp.float32),
        mesh=mesh,
        scratch_types=[
            pltpu.VMEM((B_PER_W,), jnp.int32),
            pltpu.VMEM((B_PER_W,), jnp.int32),
            pltpu.VMEM((B_PER_W, SLICE), jnp.float32),
            pltpu.VMEM((B_PER_W, OUT_COLS), jnp.float32),
            pltpu.SemaphoreType.DMA,
        ],
        compiler_params=pltpu.CompilerParams(
            needs_layout_passes=False,
            disable_bounds_checks=True,
            skip_device_barrier=True,
        ),
    )


def kernel(token_ids, weight_float):
    ids32 = token_ids.astype(jnp.int32)
    out = _build()(ids32, weight_float.reshape(NSLICE, SLICE))
    return out.reshape(BATCH, EMBED, BITS)
